# baseline (speedup 1.0000x reference)
import functools
import math

import jax
import jax.numpy as jnp
from jax import lax
from jax.experimental import pallas as pl
from jax.experimental.pallas import tpu as pltpu
from jax.experimental.pallas import tpu_sc as plsc

B = 1024
L = 50
NUM_CONT = 16
N_EMB = 64
NUM_FIELDS = 26
VOCAB = 100000
M_CONST = 10000

GROUPS = (7, 7, 7, 5)
N_TOK = B * L
T_CHUNK = 8
N_CHUNKS = N_TOK // T_CHUNK
NW = 32
CPW = N_CHUNKS // NW
NB = 5
CPB = CPW // NB
TPB = CPB * T_CHUNK
TPW = CPW * T_CHUNK

R_BLK = 6400
NVB = 16
VOCAB_PAD = R_BLK * NVB


def _repack_body(in_ref, out_ref):
    x = in_ref[0].T
    out_ref[...] = jnp.concatenate([x, x], axis=1)


def _repack(tables_t, field0, nfields):
    grid = (nfields, NVB)
    return pl.pallas_call(
        _repack_body,
        grid=grid,
        in_specs=[pl.BlockSpec((1, N_EMB, R_BLK),
                               lambda i, j: (i + field0, 0, j))],
        out_specs=pl.BlockSpec(
            (R_BLK, 2 * N_EMB),
            lambda i, j: (i * NVB + j, 0)),
        out_shape=jax.ShapeDtypeStruct((nfields * VOCAB_PAD, 2 * N_EMB),
                                       jnp.float32),
    )(tables_t)


def _sc_gather_sum(tab128, idx2d, nfields):
    mesh = plsc.VectorSubcoreMesh(core_axis_name="c", subcore_axis_name="s")
    idx_per_chunk = T_CHUNK * nfields

    @functools.partial(
        pl.kernel,
        out_type=jax.ShapeDtypeStruct((N_TOK, N_EMB), jnp.float32),
        mesh=mesh,
        scratch_types=[
            pltpu.VMEM((CPB, idx_per_chunk), jnp.int32),
            pltpu.VMEM((idx_per_chunk, 2 * N_EMB), jnp.float32),
            pltpu.VMEM((idx_per_chunk, 2 * N_EMB), jnp.float32),
            pltpu.VMEM((TPB, N_EMB), jnp.float32),
            pltpu.SemaphoreType.DMA,
            pltpu.SemaphoreType.DMA,
        ],
    )
    def k(tab_hbm, idx_hbm, out_hbm, idx_v, rows0, rows1, out_v, sem0, sem1):
        nc = 2
        wid = lax.axis_index("s") * nc + lax.axis_index("c")

        def accumulate(rows, c):
            base = c * T_CHUNK
            for t in range(T_CHUNK):
                for j in range(N_EMB // 16):
                    acc = rows[t * nfields, pl.ds(j * 16, 16)]
                    for f in range(1, nfields):
                        acc = acc + rows[t * nfields + f, pl.ds(j * 16, 16)]
                    out_v[base + t, pl.ds(j * 16, 16)] = acc

        def start(buf, sem, c):
            pltpu.make_async_copy(tab_hbm.at[idx_v.at[c]], buf, sem).start()

        def wait(buf, sem):
            pltpu.make_async_copy(tab_hbm.at[idx_v.at[0]], buf, sem).wait()

        def blk_body(blk, _):
            chunk0 = wid * CPW + blk * CPB
            pltpu.sync_copy(idx_hbm.at[pl.ds(chunk0, CPB)], idx_v)
            start(rows0, sem0, 0)
            start(rows1, sem1, 1)

            def body(i, _):
                c0 = 2 * i
                wait(rows0, sem0)
                accumulate(rows0, c0)

                @pl.when(c0 + 2 < CPB)
                def _():
                    start(rows0, sem0, c0 + 2)

                wait(rows1, sem1)
                accumulate(rows1, c0 + 1)

                @pl.when(c0 + 3 < CPB)
                def _():
                    start(rows1, sem1, c0 + 3)

                return 0

            lax.fori_loop(0, CPB // 2, body, 0)
            pltpu.sync_copy(out_v, out_hbm.at[pl.ds(wid * TPW + blk * TPB, TPB)])
            return 0

        lax.fori_loop(0, NB, blk_body, 0)

    return k(tab128, idx2d)


BT = 1600


def _tc_body(cont_ref, ca_ref, cb_ref, cc_ref, cd_ref, pe_ref, w_ref, b_ref,
             g_ref, be_ref, out_ref):
    x = cont_ref[...]
    ce = jnp.dot(x, w_ref[...], preferred_element_type=jnp.float32)
    ce = ce + b_ref[...]

    comb = (ce + ca_ref[...] + cb_ref[...] + cc_ref[...] + cd_ref[...]
            + pe_ref[...])
    mu = jnp.mean(comb, axis=1, keepdims=True)
    d = comb - mu
    var = jnp.mean(d * d, axis=1, keepdims=True)
    y = d * lax.rsqrt(var + 1e-12) * g_ref[...] + be_ref[...]
    out_ref[...] = y.reshape(BT // L, L, N_EMB)


def _tc_dense(cont2d, cats, pe_tile, W, b, gamma, beta):
    grid = (N_TOK // BT,)
    cat_spec = pl.BlockSpec((BT, N_EMB), lambda i: (i, 0))
    return pl.pallas_call(
        _tc_body,
        grid=grid,
        in_specs=[
            pl.BlockSpec((BT, NUM_CONT), lambda i: (i, 0)),
            cat_spec, cat_spec, cat_spec, cat_spec,
            pl.BlockSpec((BT, N_EMB), lambda i: (0, 0)),
            pl.BlockSpec((NUM_CONT, N_EMB), lambda i: (0, 0)),
            pl.BlockSpec((1, N_EMB), lambda i: (0, 0)),
            pl.BlockSpec((1, N_EMB), lambda i: (0, 0)),
            pl.BlockSpec((1, N_EMB), lambda i: (0, 0)),
        ],
        out_specs=pl.BlockSpec((BT // L, L, N_EMB), lambda i: (i, 0, 0)),
        out_shape=jax.ShapeDtypeStruct((B, L, N_EMB), jnp.float32),
    )(cont2d, *cats, pe_tile, W, b.reshape(1, N_EMB), gamma.reshape(1, N_EMB),
      beta.reshape(1, N_EMB))


def _pos_emb_tile():
    half = N_EMB // 2
    freqs = jnp.exp(jnp.arange(half, dtype=jnp.float32)
                    * (-math.log(M_CONST) / half))
    ang = jnp.arange(L, dtype=jnp.float32)[:, None] * freqs[None, :]
    pe = jnp.concatenate([jnp.sin(ang), jnp.cos(ang)], axis=-1)
    return jnp.tile(pe, (BT // L, 1))


def kernel(continuous_data, categorical_data, W, b, tables, gamma, beta):
    tab_t = jnp.transpose(tables, (0, 2, 1))

    idxs = []
    f0 = 0
    for nf in GROUPS:
        offs = (jnp.arange(nf, dtype=jnp.int32) * VOCAB_PAD)[None, None, :]
        idxs.append((categorical_data[:, :, f0:f0 + nf] + offs).reshape(
            N_CHUNKS, T_CHUNK * nf))
        f0 += nf
    tab_t, *idxs = lax.optimization_barrier((tab_t, *idxs))

    cats = []
    f0 = 0
    for nf, idx_g in zip(GROUPS, idxs):
        tab_g = _repack(tab_t, f0, nf)
        cats.append(_sc_gather_sum(tab_g, idx_g, nf))
        f0 += nf

    return _tc_dense(continuous_data.reshape(N_TOK, NUM_CONT), cats,
                     _pos_emb_tile(), W, b, gamma, beta)


# speedup vs baseline: 1.0075x; 1.0075x over previous
import functools
import math

import jax
import jax.numpy as jnp
from jax import lax
from jax.experimental import pallas as pl
from jax.experimental.pallas import tpu as pltpu
from jax.experimental.pallas import tpu_sc as plsc

B = 1024
L = 50
NUM_CONT = 16
N_EMB = 64
NUM_FIELDS = 26
VOCAB = 100000
M_CONST = 10000

GROUPS = (7, 7, 7, 5)
N_TOK = B * L
T_CHUNK = 8
N_CHUNKS = N_TOK // T_CHUNK
NW = 32
CPW = N_CHUNKS // NW
NB = 5
CPB = CPW // NB
TPB = CPB * T_CHUNK
TPW = CPW * T_CHUNK

R_BLK = 6400
NVB = 16
VOCAB_PAD = R_BLK * NVB


def _repack_body(in_ref, out_ref):
    x = in_ref[0].T
    out_ref[...] = jnp.concatenate([x, x], axis=1)


def _repack(tables_t, field0, nfields):
    grid = (nfields, NVB)
    return pl.pallas_call(
        _repack_body,
        grid=grid,
        in_specs=[pl.BlockSpec((1, N_EMB, R_BLK),
                               lambda i, j: (i + field0, 0, j))],
        out_specs=pl.BlockSpec(
            (R_BLK, 2 * N_EMB),
            lambda i, j: (i * NVB + j, 0)),
        out_shape=jax.ShapeDtypeStruct((nfields * VOCAB_PAD, 2 * N_EMB),
                                       jnp.float32),
    )(tables_t)


def _sc_gather_sum(tab128, idx2d, nfields):
    mesh = plsc.VectorSubcoreMesh(core_axis_name="c", subcore_axis_name="s")
    idx_per_chunk = T_CHUNK * nfields

    @functools.partial(
        pl.kernel,
        out_type=jax.ShapeDtypeStruct((N_TOK, N_EMB), jnp.float32),
        mesh=mesh,
        scratch_types=[
            pltpu.VMEM((CPB, idx_per_chunk), jnp.int32),
            pltpu.VMEM((idx_per_chunk, 2 * N_EMB), jnp.float32),
            pltpu.VMEM((idx_per_chunk, 2 * N_EMB), jnp.float32),
            pltpu.VMEM((TPB, N_EMB), jnp.float32),
            pltpu.SemaphoreType.DMA,
            pltpu.SemaphoreType.DMA,
        ],
    )
    def k(tab_hbm, idx_hbm, out_hbm, idx_v, rows0, rows1, out_v, sem0, sem1):
        nc = 2
        wid = lax.axis_index("s") * nc + lax.axis_index("c")

        def accumulate(rows, c):
            base = c * T_CHUNK
            for t in range(T_CHUNK):
                for j in range(N_EMB // 16):
                    acc = rows[t * nfields, pl.ds(j * 16, 16)]
                    for f in range(1, nfields):
                        acc = acc + rows[t * nfields + f, pl.ds(j * 16, 16)]
                    out_v[base + t, pl.ds(j * 16, 16)] = acc

        def start(buf, sem, c):
            pltpu.make_async_copy(tab_hbm.at[idx_v.at[c]], buf, sem).start()

        def wait(buf, sem):
            pltpu.make_async_copy(tab_hbm.at[idx_v.at[0]], buf, sem).wait()

        def blk_body(blk, _):
            chunk0 = wid * CPW + blk * CPB
            pltpu.sync_copy(idx_hbm.at[pl.ds(chunk0, CPB)], idx_v)
            start(rows0, sem0, 0)
            start(rows1, sem1, 1)

            def body(i, _):
                c0 = 2 * i
                wait(rows0, sem0)
                accumulate(rows0, c0)

                @pl.when(c0 + 2 < CPB)
                def _():
                    start(rows0, sem0, c0 + 2)

                wait(rows1, sem1)
                accumulate(rows1, c0 + 1)

                @pl.when(c0 + 3 < CPB)
                def _():
                    start(rows1, sem1, c0 + 3)

                return 0

            lax.fori_loop(0, CPB // 2, body, 0)
            pltpu.sync_copy(out_v, out_hbm.at[pl.ds(wid * TPW + blk * TPB, TPB)])
            return 0

        lax.fori_loop(0, NB, blk_body, 0)

    return k(tab128, idx2d)


BT = 1600


def _tc_body(cont_ref, ca_ref, cb_ref, cc_ref, cd_ref, pe_ref, w_ref, b_ref,
             g_ref, be_ref, out_ref):
    x = cont_ref[...]
    ce = jnp.dot(x, w_ref[...], preferred_element_type=jnp.float32)
    ce = ce + b_ref[...]

    comb = (ce + ca_ref[...] + cb_ref[...] + cc_ref[...] + cd_ref[...]
            + pe_ref[...])
    mu = jnp.mean(comb, axis=1, keepdims=True)
    d = comb - mu
    var = jnp.mean(d * d, axis=1, keepdims=True)
    y = d * lax.rsqrt(var + 1e-12) * g_ref[...] + be_ref[...]
    out_ref[...] = y.reshape(BT // L, L, N_EMB)


def _tc_dense(cont2d, cats, pe_tile, W, b, gamma, beta):
    grid = (N_TOK // BT,)
    cat_spec = pl.BlockSpec((BT, N_EMB), lambda i: (i, 0))
    return pl.pallas_call(
        _tc_body,
        grid=grid,
        in_specs=[
            pl.BlockSpec((BT, NUM_CONT), lambda i: (i, 0)),
            cat_spec, cat_spec, cat_spec, cat_spec,
            pl.BlockSpec((BT, N_EMB), lambda i: (0, 0)),
            pl.BlockSpec((NUM_CONT, N_EMB), lambda i: (0, 0)),
            pl.BlockSpec((1, N_EMB), lambda i: (0, 0)),
            pl.BlockSpec((1, N_EMB), lambda i: (0, 0)),
            pl.BlockSpec((1, N_EMB), lambda i: (0, 0)),
        ],
        out_specs=pl.BlockSpec((BT // L, L, N_EMB), lambda i: (i, 0, 0)),
        out_shape=jax.ShapeDtypeStruct((B, L, N_EMB), jnp.float32),
    )(cont2d, *cats, pe_tile, W, b.reshape(1, N_EMB), gamma.reshape(1, N_EMB),
      beta.reshape(1, N_EMB))


def _pos_emb_tile():
    half = N_EMB // 2
    freqs = jnp.exp(jnp.arange(half, dtype=jnp.float32)
                    * (-math.log(M_CONST) / half))
    ang = jnp.arange(L, dtype=jnp.float32)[:, None] * freqs[None, :]
    pe = jnp.concatenate([jnp.sin(ang), jnp.cos(ang)], axis=-1)
    return jnp.tile(pe, (BT // L, 1))


def kernel(continuous_data, categorical_data, W, b, tables, gamma, beta):
    tab_t = jnp.transpose(tables, (0, 2, 1))

    cats = []
    f0 = 0
    for nf in GROUPS:
        tab_g = _repack(tab_t, f0, nf)
        offs = (jnp.arange(nf, dtype=jnp.int32) * VOCAB_PAD)[None, None, :]
        idx_g = (categorical_data[:, :, f0:f0 + nf] + offs).reshape(
            N_CHUNKS, T_CHUNK * nf)
        cats.append(_sc_gather_sum(tab_g, idx_g, nf))
        f0 += nf

    return _tc_dense(continuous_data.reshape(N_TOK, NUM_CONT), cats,
                     _pos_emb_tile(), W, b, gamma, beta)
